# baseline (speedup 1.0000x reference)
import functools

import jax
import jax.numpy as jnp
from jax import lax
from jax.experimental import pallas as pl
from jax.experimental.pallas import tpu as pltpu
from jax.experimental.pallas import tpu_sc as plsc

N = 50000
D = 256
B = 128

NT = 20480
NH = N - NT
BLK = 5000
HBLK = -(-NH // BLK)

NWT = 32
RW = NT // NWT
GR = 32
NG = RW // GR
SEG = 144


def _tc_body(idx_ref, pred_ref, tgt_ref, acc_out, cnt_out, acc_ref, cnt_ref):
    step = pl.program_id(0)

    @pl.when(step == 0)
    def _init():
        acc_ref[...] = jnp.zeros_like(acc_ref)
        cnt_ref[...] = jnp.zeros_like(cnt_ref)

    diff = pred_ref[...] - tgt_ref[...]
    sqb = (diff * diff).astype(jnp.bfloat16)
    idx = idx_ref[0, 0, :]
    row_ids = jax.lax.broadcasted_iota(jnp.int32, (B, BLK), 0)
    col_pos = jax.lax.broadcasted_iota(jnp.int32, (B, BLK), 1) + step * BLK
    ok = (row_ids == idx[None, :]) & (col_pos < NH)
    onehot_t = jnp.where(ok, jnp.float32(1), jnp.float32(0)
                         ).astype(jnp.bfloat16)
    acc_ref[...] += jnp.dot(onehot_t, sqb,
                            preferred_element_type=jnp.float32)
    cnt_ref[...] += jnp.dot(onehot_t, jnp.ones((BLK, 8), jnp.bfloat16),
                            preferred_element_type=jnp.float32)

    @pl.when(step == HBLK - 1)
    def _fini():
        acc_out[...] = acc_ref[...]
        cnt_out[...] = cnt_ref[...]


def _tc_head(idx3, pred, target):
    return pl.pallas_call(
        _tc_body,
        grid=(HBLK,),
        in_specs=[
            pl.BlockSpec((1, 1, BLK), lambda i: (i, 0, 0)),
            pl.BlockSpec((BLK, D), lambda i: (i, 0)),
            pl.BlockSpec((BLK, D), lambda i: (i, 0)),
        ],
        out_specs=[
            pl.BlockSpec((B, D), lambda i: (0, 0)),
            pl.BlockSpec((B, 8), lambda i: (0, 0)),
        ],
        out_shape=[
            jax.ShapeDtypeStruct((B, D), jnp.float32),
            jax.ShapeDtypeStruct((B, 8), jnp.float32),
        ],
        scratch_shapes=[
            pltpu.VMEM((B, D), jnp.float32),
            pltpu.VMEM((B, 8), jnp.float32),
        ],
    )(idx3, pred, target)


@functools.partial(
    pl.kernel,
    out_type=[
        jax.ShapeDtypeStruct((NWT, SEG, 128), jnp.float32),
        jax.ShapeDtypeStruct((NWT, SEG, 128), jnp.float32),
    ],
    mesh=plsc.VectorSubcoreMesh(core_axis_name="c", subcore_axis_name="s"),
    scratch_types=[
        pltpu.VMEM((GR, D), jnp.float32),
        pltpu.VMEM((GR, D), jnp.float32),
        pltpu.VMEM((GR, D), jnp.float32),
        pltpu.VMEM((GR, D), jnp.float32),
        pltpu.VMEM((RW,), jnp.int32),
        pltpu.VMEM((GR,), jnp.int32),
        pltpu.VMEM((GR, 128), jnp.float32),
        pltpu.VMEM((GR, 128), jnp.float32),
        pltpu.VMEM((SEG, 128), jnp.float32),
        pltpu.SemaphoreType.DMA,
        pltpu.SemaphoreType.DMA,
        pltpu.SemaphoreType.DMA,
        pltpu.SemaphoreType.DMA,
        pltpu.VMEM_SHARED((16 * SEG, 128), jnp.float32),
        pltpu.VMEM_SHARED((16 * SEG, 128), jnp.float32),
    ],
)
def _sc_tail(pred_hbm, tgt_hbm, idx_hbm, zeros_hbm, ones_hbm,
             osum_hbm, ocnt_hbm,
             pbuf0, pbuf1, tbuf0, tbuf1, idx_v, idx_cur, stage_v, ones_v,
             obuf_v, sem_p0, sem_p1, sem_t0, sem_t1, sh_sums, sh_cnts):
    cc = lax.axis_index("c")
    ss = lax.axis_index("s")
    wid = cc * 16 + ss
    start = NH + wid * RW

    pltpu.sync_copy(idx_hbm.at[wid], idx_v)
    pltpu.sync_copy(ones_hbm, ones_v)
    pltpu.sync_copy(zeros_hbm, obuf_v)
    pltpu.sync_copy(zeros_hbm.at[pl.ds(0, GR)], stage_v)

    off = ss * SEG
    pltpu.sync_copy(obuf_v, sh_sums.at[pl.ds(off, SEG)])
    pltpu.sync_copy(obuf_v, sh_cnts.at[pl.ds(off, SEG)])

    bufs = ((pbuf0, tbuf0, sem_p0, sem_t0), (pbuf1, tbuf1, sem_p1, sem_t1))

    def _issue(g, pb, tb, sp, st):
        row = start + g * GR
        pltpu.async_copy(pred_hbm.at[pl.ds(row, GR)], pb, sp)
        pltpu.async_copy(tgt_hbm.at[pl.ds(row, GR)], tb, st)

    def _wait(g, pb, tb, sp, st):
        row = start + g * GR
        pltpu.make_async_copy(pred_hbm.at[pl.ds(row, GR)], pb, sp).wait()
        pltpu.make_async_copy(tgt_hbm.at[pl.ds(row, GR)], tb, st).wait()

    _issue(0, *bufs[0])
    _issue(1, *bufs[1])

    def body(go, carry):
        for b in (0, 1):
            g = go * 2 + b
            pb, tb, sp, st = bufs[b]
            _wait(g, pb, tb, sp, st)
            for i in range(GR):
                d = pb[i, pl.ds(0, 16)] - tb[i, pl.ds(0, 16)]
                acc = d * d
                for k in range(1, D // 16):
                    sl = pl.ds(k * 16, 16)
                    d = pb[i, sl] - tb[i, sl]
                    acc = acc + d * d
                stage_v[i, pl.ds(0, 16)] = acc
            for k in range(GR // 16):
                sl = pl.ds(k * 16, 16)
                idx_cur[sl] = idx_v[pl.ds(g * GR + k * 16, 16)] + off
            pltpu.sync_copy(stage_v, sh_sums.at[idx_cur], add=True)
            pltpu.sync_copy(ones_v, sh_cnts.at[idx_cur], add=True)

            @pl.when(go * 2 + b + 2 < NG)
            def _next():
                _issue(g + 2, pb, tb, sp, st)
        return carry

    lax.fori_loop(0, NG // 2, body, 0)

    pltpu.sync_copy(sh_sums.at[pl.ds(off, SEG)], obuf_v)
    pltpu.sync_copy(obuf_v, osum_hbm.at[wid])
    pltpu.sync_copy(sh_cnts.at[pl.ds(off, SEG)], obuf_v)
    pltpu.sync_copy(obuf_v, ocnt_hbm.at[wid])


def _combine_body(acc_ref, cnth_ref, ts_ref, tc_ref, out_ref):
    seg = jnp.sum(acc_ref[...], axis=1)
    seg = seg + jnp.sum(ts_ref[...], axis=(0, 2))[:B]
    cnt = cnth_ref[:, 0] + jnp.sum(tc_ref[:, :, 0], axis=0)[:B]
    tot = jnp.sum(seg / jnp.maximum(cnt, 1.0))
    out_ref[...] = (tot / (D * B)).reshape(1, 1)


def _tc_combine(acc, cnth, tsums, tcnts):
    return pl.pallas_call(
        _combine_body,
        out_shape=jax.ShapeDtypeStruct((1, 1), jnp.float32),
    )(acc, cnth, tsums, tcnts)


def kernel(pred, target, batch_idx, batch_size):
    del batch_size
    idx32 = batch_idx.astype(jnp.int32)
    idx_head = idx32[:HBLK * BLK].reshape(HBLK, 1, BLK)
    idx_tail = idx32[NH:].reshape(NWT, RW)
    zeros = jnp.zeros((SEG, 128), jnp.float32)
    ones = jnp.ones((GR, 128), jnp.float32)
    acc, cnth = _tc_head(idx_head, pred, target)
    tsums, tcnts = _sc_tail(pred, target, idx_tail, zeros, ones)
    out = _tc_combine(acc, cnth, tsums, tcnts)
    return out[0, 0]


# speedup vs baseline: 1.0185x; 1.0185x over previous
import functools

import jax
import jax.numpy as jnp
from jax import lax
from jax.experimental import pallas as pl
from jax.experimental.pallas import tpu as pltpu
from jax.experimental.pallas import tpu_sc as plsc

N = 50000
D = 256
B = 128

NT = 20480
NH = N - NT
BLK = 5000
HBLK = -(-NH // BLK)

NWT = 32
RW = NT // NWT
GR = 32
NG = RW // GR
SEG = 144


def _tc_body(idx_ref, pred_ref, tgt_ref, acc_out, cnt_out, acc_ref, cnt_ref):
    step = pl.program_id(0)

    @pl.when(step == 0)
    def _init():
        acc_ref[...] = jnp.zeros_like(acc_ref)
        cnt_ref[...] = jnp.zeros_like(cnt_ref)

    diff = pred_ref[...] - tgt_ref[...]
    sqb = (diff * diff).astype(jnp.bfloat16)
    idx = idx_ref[0, 0, :]
    row_ids = jax.lax.broadcasted_iota(jnp.int32, (B, BLK), 0)
    col_pos = jax.lax.broadcasted_iota(jnp.int32, (B, BLK), 1) + step * BLK
    ok = (row_ids == idx[None, :]) & (col_pos < NH)
    onehot_t = jnp.where(ok, jnp.float32(1), jnp.float32(0)
                         ).astype(jnp.bfloat16)
    acc_ref[...] += jnp.dot(onehot_t, sqb,
                            preferred_element_type=jnp.float32)
    cnt_ref[...] += jnp.dot(onehot_t, jnp.ones((BLK, 8), jnp.bfloat16),
                            preferred_element_type=jnp.float32)

    @pl.when(step == HBLK - 1)
    def _fini():
        acc_out[...] = acc_ref[...]
        cnt_out[...] = cnt_ref[...]


def _tc_head(idx3, pred, target):
    return pl.pallas_call(
        _tc_body,
        grid=(HBLK,),
        in_specs=[
            pl.BlockSpec((1, 1, BLK), lambda i: (i, 0, 0)),
            pl.BlockSpec((BLK, D), lambda i: (i, 0)),
            pl.BlockSpec((BLK, D), lambda i: (i, 0)),
        ],
        out_specs=[
            pl.BlockSpec((B, D), lambda i: (0, 0)),
            pl.BlockSpec((B, 8), lambda i: (0, 0)),
        ],
        out_shape=[
            jax.ShapeDtypeStruct((B, D), jnp.float32),
            jax.ShapeDtypeStruct((B, 8), jnp.float32),
        ],
        scratch_shapes=[
            pltpu.VMEM((B, D), jnp.float32),
            pltpu.VMEM((B, 8), jnp.float32),
        ],
    )(idx3, pred, target)


@functools.partial(
    pl.kernel,
    out_type=[
        jax.ShapeDtypeStruct((NWT, SEG, 128), jnp.float32),
        jax.ShapeDtypeStruct((NWT, SEG, 128), jnp.float32),
    ],
    mesh=plsc.VectorSubcoreMesh(core_axis_name="c", subcore_axis_name="s"),
    scratch_types=[
        pltpu.VMEM((GR, D), jnp.float32),
        pltpu.VMEM((GR, D), jnp.float32),
        pltpu.VMEM((GR, D), jnp.float32),
        pltpu.VMEM((GR, D), jnp.float32),
        pltpu.VMEM((RW,), jnp.int32),
        pltpu.VMEM((2 * GR,), jnp.int32),
        pltpu.VMEM((2 * GR, 128), jnp.float32),
        pltpu.VMEM((2 * GR, 128), jnp.float32),
        pltpu.VMEM((SEG, 128), jnp.float32),
        pltpu.SemaphoreType.DMA,
        pltpu.SemaphoreType.DMA,
        pltpu.SemaphoreType.DMA,
        pltpu.SemaphoreType.DMA,
        pltpu.VMEM_SHARED((16 * SEG, 128), jnp.float32),
        pltpu.VMEM_SHARED((16 * SEG, 128), jnp.float32),
    ],
)
def _sc_tail(pred_hbm, tgt_hbm, idx_hbm, zeros_hbm, ones_hbm,
             osum_hbm, ocnt_hbm,
             pbuf0, pbuf1, tbuf0, tbuf1, idx_v, idx_cur, stage_v, ones_v,
             obuf_v, sem_p0, sem_p1, sem_t0, sem_t1, sh_sums, sh_cnts):
    cc = lax.axis_index("c")
    ss = lax.axis_index("s")
    wid = cc * 16 + ss
    start = NH + wid * RW

    pltpu.sync_copy(idx_hbm.at[wid], idx_v)
    pltpu.sync_copy(ones_hbm, ones_v)
    pltpu.sync_copy(zeros_hbm, obuf_v)
    pltpu.sync_copy(zeros_hbm.at[pl.ds(0, 2 * GR)], stage_v)

    off = ss * SEG
    pltpu.sync_copy(obuf_v, sh_sums.at[pl.ds(off, SEG)])
    pltpu.sync_copy(obuf_v, sh_cnts.at[pl.ds(off, SEG)])

    bufs = ((pbuf0, tbuf0, sem_p0, sem_t0), (pbuf1, tbuf1, sem_p1, sem_t1))

    def _issue(g, pb, tb, sp, st):
        row = start + g * GR
        pltpu.async_copy(pred_hbm.at[pl.ds(row, GR)], pb, sp)
        pltpu.async_copy(tgt_hbm.at[pl.ds(row, GR)], tb, st)

    def _wait(g, pb, tb, sp, st):
        row = start + g * GR
        pltpu.make_async_copy(pred_hbm.at[pl.ds(row, GR)], pb, sp).wait()
        pltpu.make_async_copy(tgt_hbm.at[pl.ds(row, GR)], tb, st).wait()

    _issue(0, *bufs[0])
    _issue(1, *bufs[1])

    def body(go, carry):
        for b in (0, 1):
            g = go * 2 + b
            pb, tb, sp, st = bufs[b]
            _wait(g, pb, tb, sp, st)
            for i in range(GR):
                d0 = pb[i, pl.ds(0, 16)] - tb[i, pl.ds(0, 16)]
                d1 = pb[i, pl.ds(16, 16)] - tb[i, pl.ds(16, 16)]
                acc0 = d0 * d0
                acc1 = d1 * d1
                for k in range(2, D // 16, 2):
                    sl0 = pl.ds(k * 16, 16)
                    sl1 = pl.ds(k * 16 + 16, 16)
                    e0 = pb[i, sl0] - tb[i, sl0]
                    e1 = pb[i, sl1] - tb[i, sl1]
                    acc0 = acc0 + e0 * e0
                    acc1 = acc1 + e1 * e1
                stage_v[b * GR + i, pl.ds(0, 16)] = acc0 + acc1
            if b == 1:
                for k in range(2 * GR // 16):
                    sl = pl.ds(k * 16, 16)
                    idx_cur[sl] = idx_v[pl.ds(go * 2 * GR + k * 16, 16)] + off
                pltpu.sync_copy(stage_v, sh_sums.at[idx_cur], add=True)
                pltpu.sync_copy(ones_v, sh_cnts.at[idx_cur], add=True)

            @pl.when(go * 2 + b + 2 < NG)
            def _next():
                _issue(g + 2, pb, tb, sp, st)
        return carry

    lax.fori_loop(0, NG // 2, body, 0)

    pltpu.sync_copy(sh_sums.at[pl.ds(off, SEG)], obuf_v)
    pltpu.sync_copy(obuf_v, osum_hbm.at[wid])
    pltpu.sync_copy(sh_cnts.at[pl.ds(off, SEG)], obuf_v)
    pltpu.sync_copy(obuf_v, ocnt_hbm.at[wid])


def _combine_body(acc_ref, cnth_ref, ts_ref, tc_ref, out_ref):
    seg = jnp.sum(acc_ref[...], axis=1)
    seg = seg + jnp.sum(ts_ref[...], axis=(0, 2))[:B]
    cnt = cnth_ref[:, 0] + jnp.sum(tc_ref[:, :, 0], axis=0)[:B]
    tot = jnp.sum(seg / jnp.maximum(cnt, 1.0))
    out_ref[...] = (tot / (D * B)).reshape(1, 1)


def _tc_combine(acc, cnth, tsums, tcnts):
    return pl.pallas_call(
        _combine_body,
        out_shape=jax.ShapeDtypeStruct((1, 1), jnp.float32),
    )(acc, cnth, tsums, tcnts)


def kernel(pred, target, batch_idx, batch_size):
    del batch_size
    idx32 = batch_idx.astype(jnp.int32)
    idx_head = idx32[:HBLK * BLK].reshape(HBLK, 1, BLK)
    idx_tail = idx32[NH:].reshape(NWT, RW)
    zeros = jnp.zeros((SEG, 128), jnp.float32)
    ones = jnp.ones((2 * GR, 128), jnp.float32)
    acc, cnth = _tc_head(idx_head, pred, target)
    tsums, tcnts = _sc_tail(pred, target, idx_tail, zeros, ones)
    out = _tc_combine(acc, cnth, tsums, tcnts)
    return out[0, 0]


# speedup vs baseline: 2.2425x; 2.2017x over previous
import functools

import jax
import jax.numpy as jnp
from jax import lax
from jax.experimental import pallas as pl
from jax.experimental.pallas import tpu as pltpu
from jax.experimental.pallas import tpu_sc as plsc

N = 50000
D = 256
B = 128
BLK = 5000
NBLK = N // BLK

NW = 16
NJ = 25
LW = 128
PW = NJ * LW
NP = NW * PW
SEG = 144


def _tc_body(idx_ref, pred_ref, tgt_ref, out_ref, acc_ref):
    step = pl.program_id(0)

    @pl.when(step == 0)
    def _init():
        acc_ref[...] = jnp.zeros_like(acc_ref)

    diff = pred_ref[...] - tgt_ref[...]
    sqb = (diff * diff).astype(jnp.bfloat16)
    idx = idx_ref[0, 0, :]
    row_ids = jax.lax.broadcasted_iota(jnp.int32, (B, BLK), 0)
    onehot_t = jnp.where(row_ids == idx[None, :],
                         jnp.float32(1), jnp.float32(0)
                         ).astype(jnp.bfloat16)
    acc_ref[...] += jnp.dot(onehot_t, sqb,
                            preferred_element_type=jnp.float32)

    @pl.when(step == NBLK - 1)
    def _fini():
        out_ref[...] = acc_ref[...]


def _tc_seg_sums(idx3, pred, target):
    return pl.pallas_call(
        _tc_body,
        grid=(NBLK,),
        in_specs=[
            pl.BlockSpec((1, 1, BLK), lambda i: (i, 0, 0)),
            pl.BlockSpec((BLK, D), lambda i: (i, 0)),
            pl.BlockSpec((BLK, D), lambda i: (i, 0)),
        ],
        out_specs=pl.BlockSpec((B, D), lambda i: (0, 0)),
        out_shape=jax.ShapeDtypeStruct((B, D), jnp.float32),
        scratch_shapes=[pltpu.VMEM((B, D), jnp.float32)],
    )(idx3, pred, target)


@functools.partial(
    pl.kernel,
    out_type=jax.ShapeDtypeStruct((B,), jnp.float32),
    mesh=plsc.VectorSubcoreMesh(
        core_axis_name="c", subcore_axis_name="s", num_cores=1),
    scratch_types=[
        pltpu.VMEM((NJ, LW), jnp.int32),
        pltpu.VMEM((LW,), jnp.float32),
        pltpu.VMEM((SEG,), jnp.float32),
        pltpu.VMEM((NW * SEG,), jnp.float32),
        pltpu.VMEM((B,), jnp.float32),
        pltpu.VMEM_SHARED((NW * SEG,), jnp.float32),
    ],
)
def _sc_counts(idx_hbm, out_hbm, idx_v, ones_v, zero_v, red_v, out_v,
               sh_cnts):
    w = lax.axis_index("s")
    pltpu.sync_copy(idx_hbm.at[w], idx_v)

    for k in range(LW // 16):
        ones_v[pl.ds(k * 16, 16)] = jnp.ones((16,), jnp.float32)
    for j in range(SEG // 16):
        zero_v[pl.ds(j * 16, 16)] = jnp.zeros((16,), jnp.float32)

    off = w * SEG
    pltpu.sync_copy(zero_v, sh_cnts.at[pl.ds(off, SEG)])

    for j in range(NJ):
        for k in range(LW // 16):
            sl = pl.ds(k * 16, 16)
            idx_v[j, sl] = idx_v[j, sl] + off

    for j in range(NJ):
        pltpu.sync_copy(ones_v, sh_cnts.at[idx_v.at[j]], add=True)

    plsc.subcore_barrier()

    @pl.when(w == 0)
    def _finish():
        pltpu.sync_copy(sh_cnts, red_v)
        for j in range(B // 16):
            c_j = jnp.zeros((16,), jnp.float32)
            for ww in range(NW):
                c_j = c_j + red_v[pl.ds(ww * SEG + j * 16, 16)]
            out_v[pl.ds(j * 16, 16)] = c_j
        pltpu.sync_copy(out_v, out_hbm)


def _combine_body(acc_ref, cnt_ref, out_ref):
    seg = jnp.sum(acc_ref[...], axis=1)
    cnt = cnt_ref[...]
    tot = jnp.sum(seg / jnp.maximum(cnt, 1.0))
    out_ref[...] = (tot / (D * B)).reshape(1, 1)


def _tc_combine(acc, cnt):
    return pl.pallas_call(
        _combine_body,
        out_shape=jax.ShapeDtypeStruct((1, 1), jnp.float32),
    )(acc, cnt)


def kernel(pred, target, batch_idx, batch_size):
    del batch_size
    idx32 = batch_idx.astype(jnp.int32)
    idx3 = idx32.reshape(NBLK, 1, BLK)
    idx_pad = jnp.concatenate(
        [idx32, jnp.full((NP - N,), B, jnp.int32)]).reshape(NW, NJ, LW)
    acc = _tc_seg_sums(idx3, pred, target)
    cnt = _sc_counts(idx_pad)
    out = _tc_combine(acc, cnt)
    return out[0, 0]


# speedup vs baseline: 2.3007x; 1.0259x over previous
import functools

import jax
import jax.numpy as jnp
from jax import lax
from jax.experimental import pallas as pl
from jax.experimental.pallas import tpu as pltpu
from jax.experimental.pallas import tpu_sc as plsc

N = 50000
D = 256
B = 128
BLK = 5000
NBLK = N // BLK

NW = 16
NJ = 25
LW = 128
PW = NJ * LW
NP = NW * PW
LAST = N - (NW - 1) * PW
SEG = 144


def _tc_body(idx_ref, pred_ref, tgt_ref, out_ref, acc_ref):
    step = pl.program_id(0)

    @pl.when(step == 0)
    def _init():
        acc_ref[...] = jnp.zeros_like(acc_ref)

    diff = pred_ref[...] - tgt_ref[...]
    sqb = (diff * diff).astype(jnp.bfloat16)
    idx = idx_ref[0, 0, :]
    row_ids = jax.lax.broadcasted_iota(jnp.int32, (B, BLK), 0)
    onehot_t = jnp.where(row_ids == idx[None, :],
                         jnp.float32(1), jnp.float32(0)
                         ).astype(jnp.bfloat16)
    acc_ref[...] += jnp.dot(onehot_t, sqb,
                            preferred_element_type=jnp.float32)

    @pl.when(step == NBLK - 1)
    def _fini():
        out_ref[...] = acc_ref[...]


def _tc_seg_sums(idx3, pred, target):
    return pl.pallas_call(
        _tc_body,
        grid=(NBLK,),
        in_specs=[
            pl.BlockSpec((1, 1, BLK), lambda i: (i, 0, 0)),
            pl.BlockSpec((BLK, D), lambda i: (i, 0)),
            pl.BlockSpec((BLK, D), lambda i: (i, 0)),
        ],
        out_specs=pl.BlockSpec((B, D), lambda i: (0, 0)),
        out_shape=jax.ShapeDtypeStruct((B, D), jnp.float32),
        scratch_shapes=[pltpu.VMEM((B, D), jnp.float32)],
    )(idx3, pred, target)


@functools.partial(
    pl.kernel,
    out_type=jax.ShapeDtypeStruct((B,), jnp.float32),
    mesh=plsc.VectorSubcoreMesh(
        core_axis_name="c", subcore_axis_name="s", num_cores=1),
    scratch_types=[
        pltpu.VMEM((PW,), jnp.int32),
        pltpu.VMEM((LW,), jnp.int32),
        pltpu.VMEM((LW,), jnp.float32),
        pltpu.VMEM((SEG,), jnp.float32),
        pltpu.VMEM((NW * SEG,), jnp.float32),
        pltpu.VMEM((B,), jnp.float32),
        pltpu.VMEM_SHARED((NW * SEG,), jnp.float32),
    ],
)
def _sc_counts(idx_hbm, out_hbm, idx_v, idx_cur, ones_v, zero_v, red_v,
               out_v, sh_cnts):
    w = lax.axis_index("s")
    base = w * PW

    pad16 = jnp.full((16,), B, jnp.int32)
    for j in range((PW - LAST) // 16):
        idx_v[pl.ds(LAST + j * 16, 16)] = pad16

    @pl.when(w < NW - 1)
    def _full():
        pltpu.sync_copy(idx_hbm.at[pl.ds(base, PW)], idx_v)

    @pl.when(w == NW - 1)
    def _part():
        pltpu.sync_copy(idx_hbm.at[pl.ds(base, LAST)],
                        idx_v.at[pl.ds(0, LAST)])

    for k in range(LW // 16):
        ones_v[pl.ds(k * 16, 16)] = jnp.ones((16,), jnp.float32)
    for j in range(SEG // 16):
        zero_v[pl.ds(j * 16, 16)] = jnp.zeros((16,), jnp.float32)

    off = w * SEG
    pltpu.sync_copy(zero_v, sh_cnts.at[pl.ds(off, SEG)])

    for j in range(NJ):
        for k in range(LW // 16):
            sl = pl.ds(k * 16, 16)
            idx_cur[sl] = idx_v[pl.ds(j * LW + k * 16, 16)] + off
        pltpu.sync_copy(ones_v, sh_cnts.at[idx_cur], add=True)

    plsc.subcore_barrier()

    @pl.when(w == 0)
    def _finish():
        pltpu.sync_copy(sh_cnts, red_v)
        for j in range(B // 16):
            c_j = jnp.zeros((16,), jnp.float32)
            for ww in range(NW):
                c_j = c_j + red_v[pl.ds(ww * SEG + j * 16, 16)]
            out_v[pl.ds(j * 16, 16)] = c_j
        pltpu.sync_copy(out_v, out_hbm)


def _combine_body(acc_ref, cnt_ref, out_ref):
    seg = jnp.sum(acc_ref[...], axis=1)
    cnt = cnt_ref[...]
    tot = jnp.sum(seg / jnp.maximum(cnt, 1.0))
    out_ref[...] = (tot / (D * B)).reshape(1, 1)


def _tc_combine(acc, cnt):
    return pl.pallas_call(
        _combine_body,
        out_shape=jax.ShapeDtypeStruct((1, 1), jnp.float32),
    )(acc, cnt)


def kernel(pred, target, batch_idx, batch_size):
    del batch_size
    idx32 = batch_idx.astype(jnp.int32)
    idx3 = idx32.reshape(NBLK, 1, BLK)
    acc = _tc_seg_sums(idx3, pred, target)
    cnt = _sc_counts(idx32)
    out = _tc_combine(acc, cnt)
    return out[0, 0]
